# baseline (speedup 1.0000x reference)
import functools

import jax
import jax.numpy as jnp
from jax import lax
from jax.experimental import pallas as pl
from jax.experimental.pallas import tpu as pltpu
from jax.experimental.pallas import tpu_sc as plsc


def _gather_rows_sc(input_ids, embed):
    V, D = embed.shape
    (B,) = input_ids.shape
    info = plsc.get_sparse_core_info()
    nw = info.num_cores * info.num_subcores
    b_per_w = B // nw

    mesh = plsc.VectorSubcoreMesh(core_axis_name="c", subcore_axis_name="s")

    @functools.partial(
        pl.kernel,
        out_type=jax.ShapeDtypeStruct((B, D), jnp.float32),
        mesh=mesh,
        scratch_types=[
            pltpu.VMEM((b_per_w,), jnp.int32),
            pltpu.VMEM((b_per_w, D), jnp.float32),
            pltpu.SemaphoreType.DMA,
        ],
        compiler_params=pltpu.CompilerParams(use_tc_tiling_on_sc=False),
    )
    def gather_kernel(idx_hbm, table_hbm, out_hbm, idx_v, rows_v, sem):
        wid = lax.axis_index("s") * info.num_cores + lax.axis_index("c")
        base = wid * b_per_w
        pltpu.sync_copy(idx_hbm.at[pl.ds(base, b_per_w)], idx_v)
        pltpu.async_copy(table_hbm.at[idx_v], rows_v, sem).wait()
        pltpu.sync_copy(rows_v, out_hbm.at[pl.ds(base, b_per_w)])

    return gather_kernel(input_ids.astype(jnp.int32), embed)


def _head_tc(h, W, b, block_v=2048, nbuf=3):
    B, D = h.shape
    V = W.shape[0]
    nfull = V // block_v
    tail = V - nfull * block_v
    grid_n = nfull + (1 if tail else 0)
    nbands = B // 8
    b2 = b.reshape(1, V)
    dimnums = (((1,), (1,)), ((), ()))

    def mm_kernel(h_ref, w_ref, b_ref, out_hbm, bufs, tail_buf, sems, tail_sem):
        i = pl.program_id(0)
        slot = lax.rem(i, nbuf)

        @pl.when(jnp.logical_and(i >= nbuf, i < nfull))
        def _():
            col = pl.multiple_of((i - nbuf) * block_v, 128)
            pltpu.make_async_copy(
                bufs.at[slot], out_hbm.at[:, pl.ds(col, block_v)], sems.at[slot]
            ).wait()

        acc = lax.dot_general(
            h_ref[...], w_ref[...], dimnums, preferred_element_type=jnp.float32
        ) + jnp.broadcast_to(b_ref[...], (B, block_v))

        @pl.when(i < nfull)
        def _():
            bufs[slot] = acc
            col = pl.multiple_of(i * block_v, 128)
            for r in range(nbands):
                pltpu.make_async_copy(
                    bufs.at[slot, pl.ds(8 * r, 8), :],
                    out_hbm.at[pl.ds(8 * r, 8), pl.ds(col, block_v)],
                    sems.at[slot],
                ).start()

        if tail:
            @pl.when(i == nfull)
            def _():
                tail_buf[...] = acc[:, :tail]
                for r in range(nbands):
                    pltpu.make_async_copy(
                        tail_buf.at[pl.ds(8 * r, 8), :],
                        out_hbm.at[pl.ds(8 * r, 8), pl.ds(nfull * block_v, tail)],
                        tail_sem,
                    ).start()

        @pl.when(i == grid_n - 1)
        def _():
            for j in range(max(0, nfull - nbuf), nfull):
                pltpu.make_async_copy(
                    bufs.at[j % nbuf],
                    out_hbm.at[:, pl.ds(j * block_v, block_v)],
                    sems.at[j % nbuf],
                ).wait()
            if tail:
                pltpu.make_async_copy(
                    tail_buf, out_hbm.at[:, pl.ds(nfull * block_v, tail)], tail_sem
                ).wait()

    return pl.pallas_call(
        mm_kernel,
        grid=(grid_n,),
        in_specs=[
            pl.BlockSpec((B, D), lambda i: (0, 0)),
            pl.BlockSpec((block_v, D), lambda i: (i, 0)),
            pl.BlockSpec((1, block_v), lambda i: (0, i)),
        ],
        out_specs=pl.BlockSpec(memory_space=pl.ANY),
        out_shape=jax.ShapeDtypeStruct((B, V), jnp.float32),
        scratch_shapes=[
            pltpu.VMEM((nbuf, B, block_v), jnp.float32),
            pltpu.VMEM((B, tail if tail else block_v), jnp.float32),
            pltpu.SemaphoreType.DMA((nbuf,)),
            pltpu.SemaphoreType.DMA,
        ],
    )(h, W, b2)


def kernel(input_ids, embed, W, b):
    h = _gather_rows_sc(input_ids, embed)
    return _head_tc(h, W, b)


# speedup vs baseline: 2.8309x; 2.8309x over previous
import functools

import jax
import jax.numpy as jnp
from jax import lax
from jax.experimental import pallas as pl
from jax.experimental.pallas import tpu as pltpu
from jax.experimental.pallas import tpu_sc as plsc


def _gather_rows_sc(input_ids, embed):
    V, D = embed.shape
    (B,) = input_ids.shape
    info = plsc.get_sparse_core_info()
    nw = info.num_cores * info.num_subcores
    b_per_w = B // nw

    mesh = plsc.VectorSubcoreMesh(core_axis_name="c", subcore_axis_name="s")

    @functools.partial(
        pl.kernel,
        out_type=jax.ShapeDtypeStruct((B, D), jnp.float32),
        mesh=mesh,
        scratch_types=[
            pltpu.VMEM((b_per_w,), jnp.int32),
            pltpu.VMEM((b_per_w, D), jnp.float32),
            pltpu.SemaphoreType.DMA,
        ],
        compiler_params=pltpu.CompilerParams(use_tc_tiling_on_sc=False),
    )
    def gather_kernel(idx_hbm, table_hbm, out_hbm, idx_v, rows_v, sem):
        wid = lax.axis_index("s") * info.num_cores + lax.axis_index("c")
        base = wid * b_per_w
        pltpu.sync_copy(idx_hbm.at[pl.ds(base, b_per_w)], idx_v)
        pltpu.async_copy(table_hbm.at[idx_v], rows_v, sem).wait()
        pltpu.sync_copy(rows_v, out_hbm.at[pl.ds(base, b_per_w)])

    return gather_kernel(input_ids.astype(jnp.int32), embed)


def _head_tc(h, W, b, block_v=2048):
    B, D = h.shape
    V = W.shape[0]
    Wt = W.T
    b2 = b.reshape(1, V)
    grid_n = pl.cdiv(V, block_v)

    def mm_kernel(h_ref, wt_ref, b_ref, out_ref):
        haug = jnp.concatenate(
            [h_ref[...], jnp.ones((B, 1), jnp.float32)], axis=1
        )
        waug = jnp.concatenate([wt_ref[...], b_ref[...]], axis=0)
        out_ref[...] = lax.dot_general(
            waug,
            haug,
            dimension_numbers=(((0,), (1,)), ((), ())),
            preferred_element_type=jnp.float32,
        )

    out_t = pl.pallas_call(
        mm_kernel,
        grid=(grid_n,),
        in_specs=[
            pl.BlockSpec((B, D), lambda i: (0, 0)),
            pl.BlockSpec((D, block_v), lambda i: (0, i)),
            pl.BlockSpec((1, block_v), lambda i: (0, i)),
        ],
        out_specs=pl.BlockSpec((block_v, B), lambda i: (i, 0)),
        out_shape=jax.ShapeDtypeStruct((V, B), jnp.float32),
    )(h, Wt, b2)
    return out_t.T


def kernel(input_ids, embed, W, b):
    h = _gather_rows_sc(input_ids, embed)
    return _head_tc(h, W, b)


# speedup vs baseline: 3.2256x; 1.1394x over previous
import functools

import jax
import jax.numpy as jnp
from jax import lax
from jax.experimental import pallas as pl
from jax.experimental.pallas import tpu as pltpu
from jax.experimental.pallas import tpu_sc as plsc


def _gather_rows_sc(input_ids, embed):
    V, D = embed.shape
    (B,) = input_ids.shape
    embed_t = embed.T
    info = plsc.get_sparse_core_info()
    nw = info.num_cores * info.num_subcores
    b_per_w = B // nw

    mesh = plsc.VectorSubcoreMesh(core_axis_name="c", subcore_axis_name="s")

    @functools.partial(
        pl.kernel,
        out_type=jax.ShapeDtypeStruct((D, B), jnp.float32),
        mesh=mesh,
        scratch_types=[
            pltpu.VMEM((b_per_w,), jnp.int32),
            pltpu.VMEM((D, b_per_w), jnp.float32),
            pltpu.SemaphoreType.DMA,
        ],
        compiler_params=pltpu.CompilerParams(use_tc_tiling_on_sc=False),
    )
    def gather_kernel(idx_hbm, table_hbm, out_hbm, idx_v, rows_v, sem):
        wid = lax.axis_index("s") * info.num_cores + lax.axis_index("c")
        base = wid * b_per_w
        pltpu.sync_copy(idx_hbm.at[pl.ds(base, b_per_w)], idx_v)

        def body(f, carry):
            pltpu.make_async_copy(
                table_hbm.at[f].at[idx_v], rows_v.at[f], sem
            ).start()
            return carry

        lax.fori_loop(0, D, body, 0)
        pltpu.make_async_copy(
            table_hbm.at[:, pl.ds(0, b_per_w)], rows_v, sem
        ).wait()
        pltpu.sync_copy(rows_v, out_hbm.at[:, pl.ds(base, b_per_w)])

    return gather_kernel(input_ids.astype(jnp.int32), embed_t)


def _head_tc(ht, W, b, block_v=2048):
    D, B = ht.shape
    V = W.shape[0]
    Wt = W.T
    b2 = b.reshape(1, V)
    grid_n = pl.cdiv(V, block_v)

    def mm_kernel(ht_ref, wt_ref, b_ref, out_ref):
        haug = jnp.concatenate(
            [ht_ref[...], jnp.ones((1, B), jnp.float32)], axis=0
        )
        waug = jnp.concatenate([wt_ref[...], b_ref[...]], axis=0)
        out_ref[...] = lax.dot_general(
            waug,
            haug,
            dimension_numbers=(((0,), (0,)), ((), ())),
            preferred_element_type=jnp.float32,
        )

    out_t = pl.pallas_call(
        mm_kernel,
        grid=(grid_n,),
        in_specs=[
            pl.BlockSpec((D, B), lambda i: (0, 0)),
            pl.BlockSpec((D, block_v), lambda i: (0, i)),
            pl.BlockSpec((1, block_v), lambda i: (0, i)),
        ],
        out_specs=pl.BlockSpec((block_v, B), lambda i: (i, 0)),
        out_shape=jax.ShapeDtypeStruct((V, B), jnp.float32),
    )(ht, Wt, b2)
    return out_t.T


def kernel(input_ids, embed, W, b):
    ht = _gather_rows_sc(input_ids, embed)
    return _head_tc(ht, W, b)
